# Initial kernel scaffold; baseline (speedup 1.0000x reference)
import jax
import jax.numpy as jnp
from jax.experimental import pallas as pl


def kernel(x, hyperedge_feature, hyperedge_index, node_coord, node_batch_idx, hyperedge_batch_idx, X_X, E_E, W_hg_node, W_hg_edge, coord_w, W_x, b_x, W_e, b_e, ln_g, ln_b, W_f, b_f):
    raise NotImplementedError("write your pallas kernel here")


# speedup vs baseline: 6.3593x; 6.3593x over previous
import functools

import jax
import jax.numpy as jnp
from jax import lax
from jax.experimental import pallas as pl
from jax.experimental.pallas import tpu as pltpu
from jax.experimental.pallas import tpu_sc as plsc

F32 = jnp.float32
N = 10000
E = 320000
D = 128
NG = 16
LAMBD = 0.005

NP = 10240
EP = 327680
EPW = EP // 32
CH = 128
NCHUNK = EPW // CH
STRIPE = NP // 16

_MESH = dict(core_axis_name="c", subcore_axis_name="s", num_cores=2,
             num_subcores=16)


def _worker(cid, sid):
    return cid * 16 + sid


_LINEAR = pltpu.CompilerParams(use_tc_tiling_on_sc=False)


def _sc_counts(idx0, idx1, idx2, idx3, table16, zeros16, ones16):
    mesh = plsc.VectorSubcoreMesh(**_MESH)
    out_ty = [jax.ShapeDtypeStruct((2, NP, 16), F32) for _ in range(5)]
    scratch = [
        pltpu.VMEM((CH,), jnp.int32),
        pltpu.VMEM((CH,), jnp.int32),
        pltpu.VMEM((CH, 16), F32),
        pltpu.VMEM((CH, 16), F32),
        pltpu.VMEM_SHARED((NP, 16), F32),
        pltpu.VMEM_SHARED((NP, 16), F32),
        pltpu.VMEM_SHARED((NP, 16), F32),
        pltpu.VMEM_SHARED((NP, 16), F32),
        pltpu.VMEM_SHARED((NP, 16), F32),
        pltpu.SemaphoreType.DMA,
    ]

    @functools.partial(pl.kernel, out_type=out_ty, mesh=mesh,
                       scratch_types=scratch, compiler_params=_LINEAR)
    def body(i0, i1, i2, i3, t16, z16, o16, out0, out1, out2, out3, outA,
             iva, ivb, ones_v, rows16_v, a0, a1, a2, a3, aA, sem):
        cid = lax.axis_index("c")
        sid = lax.axis_index("s")
        r0 = pl.multiple_of(sid * STRIPE, 8)
        for a in (a0, a1, a2, a3, aA):
            pltpu.sync_copy(z16.at[pl.ds(r0, STRIPE)],
                            a.at[pl.ds(r0, STRIPE)])
        pltpu.sync_copy(o16, ones_v)
        plsc.subcore_barrier()
        base = _worker(cid, sid) * EPW

        def chunk(i, _):
            off = pl.multiple_of(base + i * CH, 8)
            pltpu.sync_copy(i0.at[pl.ds(off, CH)], iva)
            pltpu.sync_copy(ones_v, a0.at[iva], add=True)
            pltpu.sync_copy(i1.at[pl.ds(off, CH)], ivb)
            pltpu.sync_copy(ones_v, a1.at[ivb], add=True)
            pltpu.async_copy(t16.at[ivb], rows16_v, sem).wait()
            pltpu.sync_copy(rows16_v, aA.at[iva], add=True)
            pltpu.sync_copy(i2.at[pl.ds(off, CH)], iva)
            pltpu.sync_copy(ones_v, a2.at[iva], add=True)
            pltpu.sync_copy(i3.at[pl.ds(off, CH)], iva)
            pltpu.sync_copy(ones_v, a3.at[iva], add=True)
            return 0

        lax.fori_loop(0, NCHUNK, chunk, 0)
        plsc.subcore_barrier()
        for acc, out in ((a0, out0), (a1, out1), (a2, out2), (a3, out3),
                         (aA, outA)):
            pltpu.sync_copy(acc.at[pl.ds(r0, STRIPE)],
                            out.at[cid, pl.ds(r0, STRIPE)])

    return body(idx0, idx1, idx2, idx3, table16, zeros16, ones16)


def _sc_narrow(table16, src_idx, dst_idx, zeros16):
    mesh = plsc.VectorSubcoreMesh(**_MESH)
    out_ty = [jax.ShapeDtypeStruct((2, NP, 16), F32)]
    scratch = [
        pltpu.VMEM((CH,), jnp.int32),
        pltpu.VMEM((CH,), jnp.int32),
        pltpu.VMEM((CH, 16), F32),
        pltpu.VMEM_SHARED((NP, 16), F32),
        pltpu.SemaphoreType.DMA,
    ]

    @functools.partial(pl.kernel, out_type=out_ty, mesh=mesh,
                       scratch_types=scratch, compiler_params=_LINEAR)
    def body(tab, si, di, z16, out, si_v, di_v, rows_v, acc, sem):
        cid = lax.axis_index("c")
        sid = lax.axis_index("s")
        r0 = pl.multiple_of(sid * STRIPE, 8)
        pltpu.sync_copy(z16.at[pl.ds(r0, STRIPE)], acc.at[pl.ds(r0, STRIPE)])
        plsc.subcore_barrier()
        base = _worker(cid, sid) * EPW

        def chunk(i, _):
            off = pl.multiple_of(base + i * CH, 8)
            pltpu.sync_copy(si.at[pl.ds(off, CH)], si_v)
            pltpu.sync_copy(di.at[pl.ds(off, CH)], di_v)
            pltpu.async_copy(tab.at[si_v], rows_v, sem).wait()
            pltpu.sync_copy(rows_v, acc.at[di_v], add=True)
            return 0

        lax.fori_loop(0, NCHUNK, chunk, 0)
        plsc.subcore_barrier()
        pltpu.sync_copy(acc.at[pl.ds(r0, STRIPE)],
                        out.at[cid, pl.ds(r0, STRIPE)])

    return body(table16, src_idx, dst_idx, zeros16)[0]


def _sc_gather_scatter(table, src_idx, dst_idx, zeros128):
    mesh = plsc.VectorSubcoreMesh(**_MESH)
    out_ty = [jax.ShapeDtypeStruct((2, NP, D), F32)]
    scratch = [
        pltpu.VMEM((CH,), jnp.int32),
        pltpu.VMEM((CH,), jnp.int32),
        pltpu.VMEM((CH, D), F32),
        pltpu.VMEM_SHARED((NP, D), F32),
        pltpu.SemaphoreType.DMA,
    ]

    @functools.partial(pl.kernel, out_type=out_ty, mesh=mesh,
                       scratch_types=scratch)
    def body(tab, si, di, z128, out, si_v, di_v, rows_v, acc, sem):
        cid = lax.axis_index("c")
        sid = lax.axis_index("s")
        r0 = pl.multiple_of(sid * STRIPE, 8)
        pltpu.sync_copy(z128.at[pl.ds(r0, STRIPE)], acc.at[pl.ds(r0, STRIPE)])
        plsc.subcore_barrier()
        base = _worker(cid, sid) * EPW

        def chunk(i, _):
            off = pl.multiple_of(base + i * CH, 8)
            pltpu.sync_copy(si.at[pl.ds(off, CH)], si_v)
            pltpu.sync_copy(di.at[pl.ds(off, CH)], di_v)
            pltpu.async_copy(tab.at[si_v], rows_v, sem).wait()
            pltpu.sync_copy(rows_v, acc.at[di_v], add=True)
            return 0

        lax.fori_loop(0, NCHUNK, chunk, 0)
        plsc.subcore_barrier()
        pltpu.sync_copy(acc.at[pl.ds(r0, STRIPE)],
                        out.at[cid, pl.ds(r0, STRIPE)])

    return body(table, src_idx, dst_idx, zeros128)[0]


def _sc_gcn_pair(tab_x, xx_src, xx_dst, tab_e, ee_src, ee_dst, zeros128):
    mesh = plsc.VectorSubcoreMesh(**_MESH)
    out_ty = [jax.ShapeDtypeStruct((2, NP, D), F32)]
    epw = EP // 16
    nchunk = epw // CH
    scratch = [
        pltpu.VMEM((CH,), jnp.int32),
        pltpu.VMEM((CH,), jnp.int32),
        pltpu.VMEM((CH, D), F32),
        pltpu.VMEM_SHARED((NP, D), F32),
        pltpu.SemaphoreType.DMA,
    ]

    @functools.partial(pl.kernel, out_type=out_ty, mesh=mesh,
                       scratch_types=scratch)
    def body(tx, xs, xd, te, es, ed, z128, out, si_v, di_v, rows_v, acc, sem):
        cid = lax.axis_index("c")
        sid = lax.axis_index("s")
        r0 = pl.multiple_of(sid * STRIPE, 8)
        pltpu.sync_copy(z128.at[pl.ds(r0, STRIPE)], acc.at[pl.ds(r0, STRIPE)])
        plsc.subcore_barrier()
        base = sid * epw

        def run(tab, si, di):
            def chunk(i, _):
                off = pl.multiple_of(base + i * CH, 8)
                pltpu.sync_copy(si.at[pl.ds(off, CH)], si_v)
                pltpu.sync_copy(di.at[pl.ds(off, CH)], di_v)
                pltpu.async_copy(tab.at[si_v], rows_v, sem).wait()
                pltpu.sync_copy(rows_v, acc.at[di_v], add=True)
                return 0
            lax.fori_loop(0, nchunk, chunk, 0)

        @pl.when(cid == 0)
        def _():
            run(tx, xs, xd)

        @pl.when(cid == 1)
        def _():
            run(te, es, ed)

        plsc.subcore_barrier()
        pltpu.sync_copy(acc.at[pl.ds(r0, STRIPE)],
                        out.at[cid, pl.ds(r0, STRIPE)])

    return body(tab_x, xx_src, xx_dst, tab_e, ee_src, ee_dst, zeros128)[0]


_BM = 256
_GP = NP // _BM
_BR = 200
_GR = N // _BR


def _mm4_body(a_ref, w_ref, o_ref):
    o_ref[...] = jnp.dot(a_ref[0], w_ref[0], preferred_element_type=F32)[None]


def _mm4(a_stack, w_stack):
    return pl.pallas_call(
        _mm4_body,
        grid=(4, _GP),
        in_specs=[pl.BlockSpec((1, _BM, D), lambda b, i: (b, i, 0)),
                  pl.BlockSpec((1, D, D), lambda b, i: (b, 0, 0))],
        out_specs=pl.BlockSpec((1, _BM, D), lambda b, i: (b, i, 0)),
        out_shape=jax.ShapeDtypeStruct((4, NP, D), F32),
    )(a_stack, w_stack)


def _deg_body(cea, ceb, cna, cnb, cxa, cxb, cga, cgb, xwx, xwe,
              rde, rdn, dvx, dve, yx, ye):
    ce = cea[...] + ceb[...]
    cn = cna[...] + cnb[...]
    cx = cxa[...] + cxb[...]
    cg = cga[...] + cgb[...]
    rde[...] = 1.0 / jnp.maximum(ce, 1.0)
    rdn[...] = 1.0 / jnp.maximum(cn, 1.0)
    dx = lax.rsqrt(cx + 1.0)
    dg = lax.rsqrt(cg + 1.0)
    dvx[...] = dx
    dve[...] = dg
    yx[...] = xwx[...] * dx[:, :1]
    ye[...] = xwe[...] * dg[:, :1]


def _deg(cnts, xwx, xwe):
    s16 = pl.BlockSpec((_BM, 16), lambda i: (i, 0))
    s128 = pl.BlockSpec((_BM, D), lambda i: (i, 0))
    ins = []
    for c in cnts:
        ins += [c[0], c[1]]
    return pl.pallas_call(
        _deg_body,
        grid=(_GP,),
        in_specs=[s16] * 8 + [s128] * 2,
        out_specs=[s16, s16, s16, s16, s128, s128],
        out_shape=[jax.ShapeDtypeStruct((NP, 16), F32)] * 4
        + [jax.ShapeDtypeStruct((NP, D), F32)] * 2,
    )(*ins, xwx, xwe)


def _enew_body(ew, a1a, a1b, aAa, aAb, rde, cw, e_new, hedge16):
    en = ew[...] + (a1a[...] + a1b[...]) * rde[:, :1]
    e_new[...] = en
    ce = (aAa[...] + aAb[...]) * rde[...]
    p = jnp.dot(en, cw[...], preferred_element_type=F32)
    col = lax.broadcasted_iota(jnp.int32, (_BM, 16), 1)
    hedge16[...] = jnp.where(col == 0, p, ce * p)


def _enew(ew, acc1, accA, rde, cw):
    s16 = pl.BlockSpec((_BM, 16), lambda i: (i, 0))
    s128 = pl.BlockSpec((_BM, D), lambda i: (i, 0))
    scw = pl.BlockSpec((D, 1), lambda i: (0, 0))
    return pl.pallas_call(
        _enew_body,
        grid=(_GP,),
        in_specs=[s128, s128, s128, s16, s16, s16, scw],
        out_specs=[s128, s16],
        out_shape=[jax.ShapeDtypeStruct((NP, D), F32),
                   jax.ShapeDtypeStruct((NP, 16), F32)],
    )(ew, acc1[0], acc1[1], accA[0], accA[1], rde, cw)


def _imp_body(gx, ge, yx, ye, dvx, dve, bx, be, zi, ei):
    zi[...] = (gx[...] + yx[...]) * dvx[:, :1] + bx[...]
    ei[...] = (ge[...] + ye[...]) * dve[:, :1] + be[...]


def _imp(gcn, yx, ye, dvx, dve, bx, be):
    s16 = pl.BlockSpec((_BM, 16), lambda i: (i, 0))
    s128 = pl.BlockSpec((_BM, D), lambda i: (i, 0))
    sb = pl.BlockSpec((1, D), lambda i: (0, 0))
    return pl.pallas_call(
        _imp_body,
        grid=(_GP,),
        in_specs=[s128, s128, s128, s128, s16, s16, sb, sb],
        out_specs=[s128, s128],
        out_shape=[jax.ShapeDtypeStruct((NP, D), F32)] * 2,
    )(gcn[0], gcn[1], yx, ye, dvx, dve, bx, be)


def _post_body(x, xl, a2a, a2b, tba, tbb, rdn, c16, en, hef, g, b,
               z, z_out, e_out, uc16):
    zv = xl[...] + (a2a[...] + a2b[...]) * rdn[:, :1]
    z[...] = zv
    t = (tba[...] + tbb[...]) * rdn[...]
    uc16[...] = c16[...] * (1.0 + t[:, :1]) - t

    def lngelu(v):
        m = jnp.mean(v, axis=-1, keepdims=True)
        vc = v - m
        var = jnp.mean(vc * vc, axis=-1, keepdims=True)
        h = vc * lax.rsqrt(var + 1e-5) * g[...] + b[...]
        return 0.5 * h * (1.0 + lax.erf(h * 0.7071067811865476))

    z_out[...] = lngelu(zv) + x[...]
    e_out[...] = lngelu(en[...]) + hef[...]


def _post(x, xl, acc2, t16, rdn, c16, en, hef, g, b):
    s16 = pl.BlockSpec((_BR, 16), lambda i: (i, 0))
    s128 = pl.BlockSpec((_BR, D), lambda i: (i, 0))
    sb = pl.BlockSpec((1, D), lambda i: (0, 0))
    return pl.pallas_call(
        _post_body,
        grid=(_GR,),
        in_specs=[s128, s128, s128, s128, s16, s16, s16, s16, s128, s128,
                  sb, sb],
        out_specs=[s128, s128, s128, s16],
        out_shape=[jax.ShapeDtypeStruct((N, D), F32)] * 3
        + [jax.ShapeDtypeStruct((N, 16), F32)],
    )(x, xl, acc2[0], acc2[1], t16[0], t16[1], rdn, c16, en, hef, g, b)


def _pair_stats(z1, z2, S, G, first):
    s1 = jnp.sum(z1, axis=0, keepdims=True)
    q1 = jnp.sum(z1 * z1, axis=0, keepdims=True)
    s2 = jnp.sum(z2, axis=0, keepdims=True)
    q2 = jnp.sum(z2 * z2, axis=0, keepdims=True)
    st = jnp.concatenate([s1, q1, s2, q2], axis=0)
    g = lax.dot_general(z1, z2, (((0,), (0,)), ((), ())),
                        preferred_element_type=F32)
    S[...] = jnp.where(first, st, S[...] + st)
    G[...] = jnp.where(first, g, G[...] + g)


def _pair_loss(S, G):
    bsz = float(N)
    m1 = S[0:1] / bsz
    m2 = S[2:3] / bsz
    v10 = S[1:2] / bsz - m1 * m1
    v11 = (S[1:2] - bsz * m1 * m1) / (bsz - 1.0)
    v20 = S[3:4] / bsz - m2 * m2
    v21 = (S[3:4] - bsz * m2 * m2) / (bsz - 1.0)
    d1 = jnp.sqrt(v10 + 1e-5 * v11)
    d2 = jnp.sqrt(v20 + 1e-5 * v21)
    outer_m = lax.dot_general(m1, m2, (((0,), (0,)), ((), ())),
                              preferred_element_type=F32)
    outer_d = lax.dot_general(d1, d2, (((0,), (0,)), ((), ())),
                              preferred_element_type=F32)
    c = (G[...] - bsz * outer_m) / (bsz * outer_d)
    eye = (lax.broadcasted_iota(jnp.int32, (D, D), 0)
           == lax.broadcasted_iota(jnp.int32, (D, D), 1)).astype(F32)
    c2 = c * c
    diag_term = jnp.sum(eye * (c - 1.0) ** 2)
    return diag_term + LAMBD * (jnp.sum(c2) - jnp.sum(eye * c2))


def _gram_body(z1, z2, z3, z4, o, S1, G1, S2, G2):
    i = pl.program_id(0)
    first = i == 0
    _pair_stats(z1[...], z2[...], S1, G1, first)
    _pair_stats(z3[...], z4[...], S2, G2, first)

    @pl.when(i == _GR - 1)
    def _():
        bt = _pair_loss(S1, G1) + _pair_loss(S2, G2)
        o[...] = bt * jnp.ones((1, 1), F32)


def _gram(z1, z2, z3, z4):
    s128 = pl.BlockSpec((_BR, D), lambda i: (i, 0))
    return pl.pallas_call(
        _gram_body,
        grid=(_GR,),
        in_specs=[s128] * 4,
        out_specs=pl.BlockSpec((1, 1), lambda i: (0, 0)),
        out_shape=jax.ShapeDtypeStruct((1, 1), F32),
        scratch_shapes=[pltpu.VMEM((4, D), F32), pltpu.VMEM((D, D), F32),
                        pltpu.VMEM((4, D), F32), pltpu.VMEM((D, D), F32)],
    )(z1, z2, z3, z4)


def _pool_body(zo, eo, nb, hb, wf, bf, o, zacc, eacc, zcnt, ecnt):
    i = pl.program_id(0)
    first = i == 0
    gio = lax.broadcasted_iota(jnp.int32, (NG, _BR), 0)
    mz = (gio == nb[0]).astype(F32)
    me = (gio == hb[0]).astype(F32)
    zs = jnp.dot(mz, zo[...], preferred_element_type=F32)
    es = jnp.dot(me, eo[...], preferred_element_type=F32)
    zc = jnp.sum(mz, axis=1, keepdims=True) * jnp.ones((NG, D), F32)
    ec = jnp.sum(me, axis=1, keepdims=True) * jnp.ones((NG, D), F32)
    zacc[...] = jnp.where(first, zs, zacc[...] + zs)
    eacc[...] = jnp.where(first, es, eacc[...] + es)
    zcnt[...] = jnp.where(first, zc, zcnt[...] + zc)
    ecnt[...] = jnp.where(first, ec, ecnt[...] + ec)

    @pl.when(i == _GR - 1)
    def _():
        zg = zacc[...] / jnp.maximum(zcnt[...], 1.0)
        eg = eacc[...] / jnp.maximum(ecnt[...], 1.0)
        o[...] = (jnp.dot(zg, wf[0], preferred_element_type=F32)
                  + jnp.dot(eg, wf[1], preferred_element_type=F32) + bf[...])


def _pool(z_out, e_out, nb3, hb3, wf2, bf):
    s128 = pl.BlockSpec((_BR, D), lambda i: (i, 0))
    sidx = pl.BlockSpec((1, 1, _BR), lambda i: (i, 0, 0))
    return pl.pallas_call(
        _pool_body,
        grid=(_GR,),
        in_specs=[s128, s128, sidx, sidx,
                  pl.BlockSpec((2, D, D), lambda i: (0, 0, 0)),
                  pl.BlockSpec((1, D), lambda i: (0, 0))],
        out_specs=pl.BlockSpec((NG, D), lambda i: (0, 0)),
        out_shape=jax.ShapeDtypeStruct((NG, D), F32),
        scratch_shapes=[pltpu.VMEM((NG, D), F32)] * 4,
    )(z_out, e_out, nb3, hb3, wf2, bf)


def kernel(x, hyperedge_feature, hyperedge_index, node_coord, node_batch_idx,
           hyperedge_batch_idx, X_X, E_E, W_hg_node, W_hg_edge, coord_w,
           W_x, b_x, W_e, b_e, ln_g, ln_b, W_f, b_f):
    i32 = jnp.int32
    padrows = NP - N
    pad_e = jnp.full((EP - E,), N, i32)

    def pad_idx(a):
        return jnp.concatenate([a.astype(i32), pad_e])

    ni = pad_idx(hyperedge_index[0])
    ei = pad_idx(hyperedge_index[1])
    xx_s = pad_idx(X_X[0])
    xx_d = pad_idx(X_X[1])
    ee_s = pad_idx(E_E[0])
    ee_d = pad_idx(E_E[1])

    xp = jnp.pad(x, ((0, padrows), (0, 0)))
    hefp = jnp.pad(hyperedge_feature, ((0, padrows), (0, 0)))
    coord16 = jnp.pad(node_coord, ((0, padrows), (1, 12)))
    z128 = jnp.zeros((NP, D), F32)
    z16 = jnp.zeros((NP, 16), F32)
    o16 = jnp.ones((CH, 16), F32)

    a_stack = jnp.stack([xp, xp, hefp, hefp])
    w_stack = jnp.stack([W_hg_node, W_x, W_e, W_hg_edge])
    mm = _mm4(a_stack, w_stack)
    xl, xwx, xwe, ew = mm[0], mm[1], mm[2], mm[3]

    *cnts, accA = _sc_counts(ei, ni, xx_d, ee_d, coord16, z16, o16)

    rde, rdn, dvx, dve, yx, ye = _deg(cnts, xwx, xwe)

    acc1 = _sc_gather_scatter(xl, ni, ei, z128)

    gcn = _sc_gcn_pair(yx, xx_s, xx_d, ye, ee_s, ee_d, z128)

    e_new, hedge16 = _enew(ew, acc1, accA, rde, coord_w)

    z_imp, e_imp = _imp(gcn, yx, ye, dvx, dve, b_x[None], b_e[None])

    acc2 = _sc_gather_scatter(e_new, ei, ni, z128)
    t16 = _sc_narrow(hedge16, ei, ni, z16)

    z, z_out, e_out, uc16 = _post(
        x, xl[:N], acc2[:, :N], t16[:, :N], rdn[:N], coord16[:N],
        e_new[:N], hyperedge_feature, ln_g[None], ln_b[None])

    bt = _gram(z_imp[:N], z, e_imp[:N], e_new[:N])

    nb3 = node_batch_idx.astype(i32).reshape(_GR, 1, _BR)
    hb3 = hyperedge_batch_idx.astype(i32).reshape(_GR, 1, _BR)
    wf2 = W_f.reshape(2, D, D)
    graph = _pool(z_out, e_out, nb3, hb3, wf2, b_f[None])

    uc = uc16[:, 1:4]
    return (z_out, e_out, hyperedge_index, uc, node_batch_idx,
            hyperedge_batch_idx, graph, bt[0, 0])
